# baseline (speedup 1.0000x reference)
import functools

import jax
import jax.numpy as jnp
from jax import lax
from jax.experimental import layout as jex_layout
from jax.experimental import pallas as pl
from jax.experimental.pallas import tpu as pltpu
from jax.experimental.pallas import tpu_sc as plsc

VOCAB = 1000000
D = 64
S = 4096
T = 200
NC = 2
NS = 16
NW = NC * NS
S_PER_W = S // NW


V_BLK = 512


def _tc_transpose_body(w_ref, o_ref):
    x = w_ref[...]
    r = lax.broadcasted_iota(jnp.int32, (D, D), 0)
    c = lax.broadcasted_iota(jnp.int32, (D, D), 1)
    eye = (r == c).astype(jnp.float32)
    y = lax.dot_general(
        x, eye, (((0,), (0,)), ((), ())), preferred_element_type=jnp.float32
    )
    o_ref[...] = jnp.concatenate([y, y], axis=1)


_tc_transpose = pl.pallas_call(
    _tc_transpose_body,
    grid=((VOCAB + V_BLK - 1) // V_BLK,),
    in_specs=[pl.BlockSpec((D, V_BLK), lambda i: (0, i))],
    out_specs=pl.BlockSpec((V_BLK, 2 * D), lambda i: (i, 0)),
    out_shape=jax.ShapeDtypeStruct((VOCAB, 2 * D), jnp.float32),
)


def _make_sc_gather():
    mesh = plsc.VectorSubcoreMesh(core_axis_name="c", subcore_axis_name="s")

    @functools.partial(
        pl.kernel,
        mesh=mesh,
        out_type=jax.ShapeDtypeStruct((S, T, 2 * D), jnp.float32),
        scratch_types=[
            pltpu.VMEM((S_PER_W, T), jnp.int32),
            pltpu.VMEM((2, T, D), jnp.float32),
            pltpu.SemaphoreType.DMA,
            pltpu.SemaphoreType.DMA,
        ],
        compiler_params=pltpu.CompilerParams(use_tc_tiling_on_sc=False),
    )
    def sc_gather(idx_hbm, table_hbm, out_hbm, idx_v, rows_v, sem_g, sem_o):
        wid = lax.axis_index("s") * NC + lax.axis_index("c")
        s0 = wid * S_PER_W

        pltpu.sync_copy(idx_hbm.at[wid], idx_v)

        pltpu.async_copy(table_hbm.at[idx_v.at[0]], rows_v.at[0], sem_g)

        def body(i, _):
            for b in range(2):
                j = 2 * i + b
                bn = (b + 1) % 2

                @pl.when(j >= 1)
                def _():
                    pltpu.make_async_copy(
                        rows_v.at[bn],
                        out_hbm.at[s0 + j - 1, :, pl.ds(0, D)],
                        sem_o,
                    ).wait()

                @pl.when(j + 1 < S_PER_W)
                def _():
                    pltpu.async_copy(
                        table_hbm.at[idx_v.at[j + 1]], rows_v.at[bn], sem_g
                    )

                pltpu.make_async_copy(
                    table_hbm.at[idx_v.at[j]], rows_v.at[b], sem_g
                ).wait()
                pltpu.async_copy(
                    rows_v.at[b], out_hbm.at[s0 + j, :, pl.ds(0, D)], sem_o
                )
            return ()

        lax.fori_loop(0, S_PER_W // 2, body, ())

        pltpu.make_async_copy(
            rows_v.at[(S_PER_W - 1) % 2],
            out_hbm.at[s0 + S_PER_W - 1, :, pl.ds(0, D)],
            sem_o,
        ).wait()

    return sc_gather


_sc_gather = _make_sc_gather()


@jax.jit
def kernel(token_ids, weight):
    w_pad = _tc_transpose(weight.T)
    table2 = w_pad.reshape(2 * VOCAB, D)
    idx = (token_ids * 2).reshape(NW, S_PER_W, T).astype(jnp.int32)
    out_wide = _sc_gather(idx, table2)
    return out_wide[:, :, :D]


# speedup vs baseline: 2.2678x; 2.2678x over previous
import functools

import jax
import jax.numpy as jnp
from jax import lax
from jax.experimental import layout as jex_layout
from jax.experimental import pallas as pl
from jax.experimental.pallas import tpu as pltpu
from jax.experimental.pallas import tpu_sc as plsc

VOCAB = 1000000
D = 64
S = 4096
T = 200
NC = 2
NS = 16
NW = NC * NS
S_PER_W = S // NW


V_BLK = 4096


def _tc_transpose_body(w_ref, o_ref):
    x = w_ref[...]
    r = lax.broadcasted_iota(jnp.int32, (D, D), 0)
    c = lax.broadcasted_iota(jnp.int32, (D, D), 1)
    eye = (r == c).astype(jnp.float32)
    y = lax.dot_general(
        x, eye, (((0,), (0,)), ((), ())), preferred_element_type=jnp.float32
    )
    o_ref[...] = jnp.concatenate([y, y], axis=1)


_tc_transpose = pl.pallas_call(
    _tc_transpose_body,
    grid=((VOCAB + V_BLK - 1) // V_BLK,),
    in_specs=[pl.BlockSpec((D, V_BLK), lambda i: (0, i))],
    out_specs=pl.BlockSpec((V_BLK, 2 * D), lambda i: (i, 0)),
    out_shape=jax.ShapeDtypeStruct((VOCAB, 2 * D), jnp.float32),
)


def _make_sc_gather():
    mesh = plsc.VectorSubcoreMesh(core_axis_name="c", subcore_axis_name="s")

    @functools.partial(
        pl.kernel,
        mesh=mesh,
        out_type=jax.ShapeDtypeStruct((S, T, 2 * D), jnp.float32),
        scratch_types=[
            pltpu.VMEM((S_PER_W, T), jnp.int32),
            pltpu.VMEM((2, T, D), jnp.float32),
            pltpu.SemaphoreType.DMA,
            pltpu.SemaphoreType.DMA,
        ],
        compiler_params=pltpu.CompilerParams(use_tc_tiling_on_sc=False),
    )
    def sc_gather(idx_hbm, table_hbm, out_hbm, idx_v, rows_v, sem_g, sem_o):
        wid = lax.axis_index("s") * NC + lax.axis_index("c")
        s0 = wid * S_PER_W

        pltpu.sync_copy(idx_hbm.at[wid], idx_v)

        pltpu.async_copy(table_hbm.at[idx_v.at[0]], rows_v.at[0], sem_g)

        def body(i, _):
            for b in range(2):
                j = 2 * i + b
                bn = (b + 1) % 2

                @pl.when(j >= 1)
                def _():
                    pltpu.make_async_copy(
                        rows_v.at[bn],
                        out_hbm.at[s0 + j - 1, :, pl.ds(0, D)],
                        sem_o,
                    ).wait()

                @pl.when(j + 1 < S_PER_W)
                def _():
                    pltpu.async_copy(
                        table_hbm.at[idx_v.at[j + 1]], rows_v.at[bn], sem_g
                    )

                pltpu.make_async_copy(
                    table_hbm.at[idx_v.at[j]], rows_v.at[b], sem_g
                ).wait()
                pltpu.async_copy(
                    rows_v.at[b], out_hbm.at[s0 + j, :, pl.ds(0, D)], sem_o
                )
            return ()

        lax.fori_loop(0, S_PER_W // 2, body, ())

        pltpu.make_async_copy(
            rows_v.at[(S_PER_W - 1) % 2],
            out_hbm.at[s0 + S_PER_W - 1, :, pl.ds(0, D)],
            sem_o,
        ).wait()

    return sc_gather


_sc_gather = _make_sc_gather()


@jax.jit
def kernel(token_ids, weight):
    w_pad = _tc_transpose(weight.T)
    table2 = w_pad.reshape(2 * VOCAB, D)
    idx = (token_ids * 2).reshape(NW, S_PER_W, T).astype(jnp.int32)
    out_wide = _sc_gather(idx, table2)
    return out_wide[:, :, :D]


# speedup vs baseline: 2.5073x; 1.1056x over previous
import functools

import jax
import jax.numpy as jnp
from jax import lax
from jax.experimental import layout as jex_layout
from jax.experimental import pallas as pl
from jax.experimental.pallas import tpu as pltpu
from jax.experimental.pallas import tpu_sc as plsc

VOCAB = 1000000
D = 64
S = 4096
T = 200
NC = 2
NS = 16
NW = NC * NS
S_PER_W = S // NW


V_BLK = 8192


def _tc_transpose_body(w_ref, o_ref):
    x = w_ref[...]
    r = lax.broadcasted_iota(jnp.int32, (D, D), 0)
    c = lax.broadcasted_iota(jnp.int32, (D, D), 1)
    eye = (r == c).astype(jnp.float32)
    y = lax.dot_general(
        x, eye, (((0,), (0,)), ((), ())), preferred_element_type=jnp.float32
    )
    o_ref[...] = jnp.concatenate([y, y], axis=1)


_tc_transpose = pl.pallas_call(
    _tc_transpose_body,
    grid=((VOCAB + V_BLK - 1) // V_BLK,),
    in_specs=[pl.BlockSpec((D, V_BLK), lambda i: (0, i))],
    out_specs=pl.BlockSpec((V_BLK, 2 * D), lambda i: (i, 0)),
    out_shape=jax.ShapeDtypeStruct((VOCAB, 2 * D), jnp.float32),
)


def _make_sc_gather():
    mesh = plsc.VectorSubcoreMesh(core_axis_name="c", subcore_axis_name="s")

    @functools.partial(
        pl.kernel,
        mesh=mesh,
        out_type=jax.ShapeDtypeStruct((S, T, 2 * D), jnp.float32),
        scratch_types=[
            pltpu.VMEM((S_PER_W, T), jnp.int32),
            pltpu.VMEM((2, T, D), jnp.float32),
            pltpu.SemaphoreType.DMA,
            pltpu.SemaphoreType.DMA,
        ],
        compiler_params=pltpu.CompilerParams(use_tc_tiling_on_sc=False),
    )
    def sc_gather(idx_hbm, table_hbm, out_hbm, idx_v, rows_v, sem_g, sem_o):
        wid = lax.axis_index("s") * NC + lax.axis_index("c")
        s0 = wid * S_PER_W

        pltpu.sync_copy(idx_hbm.at[wid], idx_v)

        pltpu.async_copy(table_hbm.at[idx_v.at[0]], rows_v.at[0], sem_g)

        def body(i, _):
            for b in range(2):
                j = 2 * i + b
                bn = (b + 1) % 2

                @pl.when(j >= 1)
                def _():
                    pltpu.make_async_copy(
                        rows_v.at[bn],
                        out_hbm.at[s0 + j - 1, :, pl.ds(0, D)],
                        sem_o,
                    ).wait()

                @pl.when(j + 1 < S_PER_W)
                def _():
                    pltpu.async_copy(
                        table_hbm.at[idx_v.at[j + 1]], rows_v.at[bn], sem_g
                    )

                pltpu.make_async_copy(
                    table_hbm.at[idx_v.at[j]], rows_v.at[b], sem_g
                ).wait()
                pltpu.async_copy(
                    rows_v.at[b], out_hbm.at[s0 + j, :, pl.ds(0, D)], sem_o
                )
            return ()

        lax.fori_loop(0, S_PER_W // 2, body, ())

        pltpu.make_async_copy(
            rows_v.at[(S_PER_W - 1) % 2],
            out_hbm.at[s0 + S_PER_W - 1, :, pl.ds(0, D)],
            sem_o,
        ).wait()

    return sc_gather


_sc_gather = _make_sc_gather()


@jax.jit
def kernel(token_ids, weight):
    w_pad = _tc_transpose(weight.T)
    table2 = w_pad.reshape(2 * VOCAB, D)
    idx = (token_ids * 2).reshape(NW, S_PER_W, T).astype(jnp.int32)
    out_wide = _sc_gather(idx, table2)
    return out_wide[:, :, :D]


# speedup vs baseline: 2.6351x; 1.0509x over previous
import functools

import jax
import jax.numpy as jnp
from jax import lax
from jax.experimental import layout as jex_layout
from jax.experimental import pallas as pl
from jax.experimental.pallas import tpu as pltpu
from jax.experimental.pallas import tpu_sc as plsc

VOCAB = 1000000
D = 64
S = 4096
T = 200
NC = 2
NS = 16
NW = NC * NS
S_PER_W = S // NW


V_BLK = 16384


def _tc_transpose_body(w_ref, o_ref):
    x = w_ref[...]
    r = lax.broadcasted_iota(jnp.int32, (D, D), 0)
    c = lax.broadcasted_iota(jnp.int32, (D, D), 1)
    eye = (r == c).astype(jnp.float32)
    y = lax.dot_general(
        x, eye, (((0,), (0,)), ((), ())), preferred_element_type=jnp.float32
    )
    o_ref[...] = jnp.concatenate([y, y], axis=1)


_tc_transpose = pl.pallas_call(
    _tc_transpose_body,
    grid=((VOCAB + V_BLK - 1) // V_BLK,),
    in_specs=[pl.BlockSpec((D, V_BLK), lambda i: (0, i))],
    out_specs=pl.BlockSpec((V_BLK, 2 * D), lambda i: (i, 0)),
    out_shape=jax.ShapeDtypeStruct((VOCAB, 2 * D), jnp.float32),
)


def _make_sc_gather():
    mesh = plsc.VectorSubcoreMesh(core_axis_name="c", subcore_axis_name="s")

    @functools.partial(
        pl.kernel,
        mesh=mesh,
        out_type=jax.ShapeDtypeStruct((S, T, 2 * D), jnp.float32),
        scratch_types=[
            pltpu.VMEM((S_PER_W, T), jnp.int32),
            pltpu.VMEM((2, T, D), jnp.float32),
            pltpu.SemaphoreType.DMA,
            pltpu.SemaphoreType.DMA,
        ],
        compiler_params=pltpu.CompilerParams(use_tc_tiling_on_sc=False),
    )
    def sc_gather(idx_hbm, table_hbm, out_hbm, idx_v, rows_v, sem_g, sem_o):
        wid = lax.axis_index("s") * NC + lax.axis_index("c")
        s0 = wid * S_PER_W

        pltpu.sync_copy(idx_hbm.at[wid], idx_v)

        pltpu.async_copy(table_hbm.at[idx_v.at[0]], rows_v.at[0], sem_g)

        def body(i, _):
            for b in range(2):
                j = 2 * i + b
                bn = (b + 1) % 2

                @pl.when(j >= 1)
                def _():
                    pltpu.make_async_copy(
                        rows_v.at[bn],
                        out_hbm.at[s0 + j - 1, :, pl.ds(0, D)],
                        sem_o,
                    ).wait()

                @pl.when(j + 1 < S_PER_W)
                def _():
                    pltpu.async_copy(
                        table_hbm.at[idx_v.at[j + 1]], rows_v.at[bn], sem_g
                    )

                pltpu.make_async_copy(
                    table_hbm.at[idx_v.at[j]], rows_v.at[b], sem_g
                ).wait()
                pltpu.async_copy(
                    rows_v.at[b], out_hbm.at[s0 + j, :, pl.ds(0, D)], sem_o
                )
            return ()

        lax.fori_loop(0, S_PER_W // 2, body, ())

        pltpu.make_async_copy(
            rows_v.at[(S_PER_W - 1) % 2],
            out_hbm.at[s0 + S_PER_W - 1, :, pl.ds(0, D)],
            sem_o,
        ).wait()

    return sc_gather


_sc_gather = _make_sc_gather()


@jax.jit
def kernel(token_ids, weight):
    w_pad = _tc_transpose(weight.T)
    table2 = w_pad.reshape(2 * VOCAB, D)
    idx = (token_ids * 2).reshape(NW, S_PER_W, T).astype(jnp.int32)
    out_wide = _sc_gather(idx, table2)
    return out_wide[:, :, :D]


# speedup vs baseline: 2.6812x; 1.0175x over previous
import functools

import jax
import jax.numpy as jnp
from jax import lax
from jax.experimental import layout as jex_layout
from jax.experimental import pallas as pl
from jax.experimental.pallas import tpu as pltpu
from jax.experimental.pallas import tpu_sc as plsc

VOCAB = 1000000
D = 64
S = 4096
T = 200
NC = 2
NS = 16
NW = NC * NS
S_PER_W = S // NW


V_BLK = 24576


def _tc_transpose_body(w_ref, o_ref):
    x = w_ref[...]
    r = lax.broadcasted_iota(jnp.int32, (D, D), 0)
    c = lax.broadcasted_iota(jnp.int32, (D, D), 1)
    eye = (r == c).astype(jnp.float32)
    y = lax.dot_general(
        x, eye, (((0,), (0,)), ((), ())), preferred_element_type=jnp.float32
    )
    o_ref[...] = jnp.concatenate([y, y], axis=1)


_tc_transpose = pl.pallas_call(
    _tc_transpose_body,
    grid=((VOCAB + V_BLK - 1) // V_BLK,),
    in_specs=[pl.BlockSpec((D, V_BLK), lambda i: (0, i))],
    out_specs=pl.BlockSpec((V_BLK, 2 * D), lambda i: (i, 0)),
    out_shape=jax.ShapeDtypeStruct((VOCAB, 2 * D), jnp.float32),
)


def _make_sc_gather():
    mesh = plsc.VectorSubcoreMesh(core_axis_name="c", subcore_axis_name="s")

    @functools.partial(
        pl.kernel,
        mesh=mesh,
        out_type=jax.ShapeDtypeStruct((S, T, 2 * D), jnp.float32),
        scratch_types=[
            pltpu.VMEM((S_PER_W, T), jnp.int32),
            pltpu.VMEM((2, T, D), jnp.float32),
            pltpu.SemaphoreType.DMA,
            pltpu.SemaphoreType.DMA,
        ],
        compiler_params=pltpu.CompilerParams(use_tc_tiling_on_sc=False),
    )
    def sc_gather(idx_hbm, table_hbm, out_hbm, idx_v, rows_v, sem_g, sem_o):
        wid = lax.axis_index("s") * NC + lax.axis_index("c")
        s0 = wid * S_PER_W

        pltpu.sync_copy(idx_hbm.at[wid], idx_v)

        pltpu.async_copy(table_hbm.at[idx_v.at[0]], rows_v.at[0], sem_g)

        def body(i, _):
            for b in range(2):
                j = 2 * i + b
                bn = (b + 1) % 2

                @pl.when(j >= 1)
                def _():
                    pltpu.make_async_copy(
                        rows_v.at[bn],
                        out_hbm.at[s0 + j - 1, :, pl.ds(0, D)],
                        sem_o,
                    ).wait()

                @pl.when(j + 1 < S_PER_W)
                def _():
                    pltpu.async_copy(
                        table_hbm.at[idx_v.at[j + 1]], rows_v.at[bn], sem_g
                    )

                pltpu.make_async_copy(
                    table_hbm.at[idx_v.at[j]], rows_v.at[b], sem_g
                ).wait()
                pltpu.async_copy(
                    rows_v.at[b], out_hbm.at[s0 + j, :, pl.ds(0, D)], sem_o
                )
            return ()

        lax.fori_loop(0, S_PER_W // 2, body, ())

        pltpu.make_async_copy(
            rows_v.at[(S_PER_W - 1) % 2],
            out_hbm.at[s0 + S_PER_W - 1, :, pl.ds(0, D)],
            sem_o,
        ).wait()

    return sc_gather


_sc_gather = _make_sc_gather()


@jax.jit
def kernel(token_ids, weight):
    w_pad = _tc_transpose(weight.T)
    table2 = w_pad.reshape(2 * VOCAB, D)
    idx = (token_ids * 2).reshape(NW, S_PER_W, T).astype(jnp.int32)
    out_wide = _sc_gather(idx, table2)
    return out_wide[:, :, :D]


# speedup vs baseline: 2.8446x; 1.0610x over previous
import functools

import jax
import jax.numpy as jnp
from jax import lax
from jax.experimental import layout as jex_layout
from jax.experimental import pallas as pl
from jax.experimental.pallas import tpu as pltpu
from jax.experimental.pallas import tpu_sc as plsc

VOCAB = 1000000
D = 64
S = 4096
T = 200
NC = 2
NS = 16
NW = NC * NS
S_PER_W = S // NW


V_BLK = 24576


def _tc_transpose_body(w_ref, o_ref):
    x = w_ref[...]
    r = lax.broadcasted_iota(jnp.int32, (D, D), 0)
    c = lax.broadcasted_iota(jnp.int32, (D, D), 1)
    eye = (r == c).astype(jnp.float32)
    y = lax.dot_general(
        x, eye, (((0,), (0,)), ((), ())), preferred_element_type=jnp.float32
    )
    o_ref[:, 0:D] = y


_tc_transpose = pl.pallas_call(
    _tc_transpose_body,
    grid=((VOCAB + V_BLK - 1) // V_BLK,),
    in_specs=[pl.BlockSpec((D, V_BLK), lambda i: (0, i))],
    out_specs=pl.BlockSpec((V_BLK, 2 * D), lambda i: (i, 0)),
    out_shape=jax.ShapeDtypeStruct((VOCAB, 2 * D), jnp.float32),
)


def _make_sc_gather():
    mesh = plsc.VectorSubcoreMesh(core_axis_name="c", subcore_axis_name="s")

    @functools.partial(
        pl.kernel,
        mesh=mesh,
        out_type=jax.ShapeDtypeStruct((S, T, 2 * D), jnp.float32),
        scratch_types=[
            pltpu.VMEM((S_PER_W, T), jnp.int32),
            pltpu.VMEM((2, T, D), jnp.float32),
            pltpu.SemaphoreType.DMA,
            pltpu.SemaphoreType.DMA,
        ],
        compiler_params=pltpu.CompilerParams(use_tc_tiling_on_sc=False),
    )
    def sc_gather(idx_hbm, table_hbm, out_hbm, idx_v, rows_v, sem_g, sem_o):
        wid = lax.axis_index("s") * NC + lax.axis_index("c")
        s0 = wid * S_PER_W

        pltpu.sync_copy(idx_hbm.at[wid], idx_v)

        pltpu.async_copy(table_hbm.at[idx_v.at[0]], rows_v.at[0], sem_g)

        def body(i, _):
            for b in range(2):
                j = 2 * i + b
                bn = (b + 1) % 2

                @pl.when(j >= 1)
                def _():
                    pltpu.make_async_copy(
                        rows_v.at[bn],
                        out_hbm.at[s0 + j - 1, :, pl.ds(0, D)],
                        sem_o,
                    ).wait()

                @pl.when(j + 1 < S_PER_W)
                def _():
                    pltpu.async_copy(
                        table_hbm.at[idx_v.at[j + 1]], rows_v.at[bn], sem_g
                    )

                pltpu.make_async_copy(
                    table_hbm.at[idx_v.at[j]], rows_v.at[b], sem_g
                ).wait()
                pltpu.async_copy(
                    rows_v.at[b], out_hbm.at[s0 + j, :, pl.ds(0, D)], sem_o
                )
            return ()

        lax.fori_loop(0, S_PER_W // 2, body, ())

        pltpu.make_async_copy(
            rows_v.at[(S_PER_W - 1) % 2],
            out_hbm.at[s0 + S_PER_W - 1, :, pl.ds(0, D)],
            sem_o,
        ).wait()

    return sc_gather


_sc_gather = _make_sc_gather()


@jax.jit
def kernel(token_ids, weight):
    w_pad = _tc_transpose(weight.T)
    table2 = w_pad.reshape(2 * VOCAB, D)
    idx = (token_ids * 2).reshape(NW, S_PER_W, T).astype(jnp.int32)
    out_wide = _sc_gather(idx, table2)
    return out_wide[:, :, :D]
